# baseline (speedup 1.0000x reference)
import jax
import jax.numpy as jnp
from jax import lax
from jax.experimental import pallas as pl
from jax.experimental.pallas import tpu as pltpu
from jax.experimental.pallas import tpu_sc as plsc
import numpy as np

NUM_PATH = 4
OUT_DIM = 128
NUM_ELEMENTS = 8
N_NODES = 100000
ALPHA = 1.0 / np.sqrt(float(NUM_PATH))

L = 16
NC, NS = 2, 16
NW = NC * NS
C = 64
NCHUNK = (N_NODES + C - 1) // C
ITERS = (NCHUNK + NW - 1) // NW
OUTER = (ITERS + 1) // 2
LAST_BASE = N_NODES - C
WSZ = NUM_ELEMENTS * OUT_DIM * NUM_PATH
XCH = C * OUT_DIM
XB = NUM_PATH * XCH
ACH = C * NUM_ELEMENTS
OCH = C * OUT_DIM


def _body(x_hbm, attrs_hbm, w_hbm, out_hbm,
          x_buf, out_buf, a_buf, w_buf, ei_buf,
          isem0, isem1, osem0, osem1):
    wid = lax.axis_index("c") * NS + lax.axis_index("s")
    pltpu.sync_copy(w_hbm, w_buf)
    iota = lax.iota(jnp.int32, L)
    in_sems = (isem0, isem1)
    out_sems = (osem0, osem1)

    def valid(i):
        return (i * NW + wid) < NCHUNK

    def chunk_base(i):
        return jnp.minimum((i * NW + wid) * C, LAST_BASE)

    def in_copies(i, b):
        base = chunk_base(i)
        sem = in_sems[b]
        cps = [pltpu.make_async_copy(
                   x_hbm.at[pl.ds(p * N_NODES * OUT_DIM + base * OUT_DIM, XCH)],
                   x_buf.at[pl.ds(b * XB + p * XCH, XCH)], sem)
               for p in range(NUM_PATH)]
        cps.append(pltpu.make_async_copy(
            attrs_hbm.at[pl.ds(base * NUM_ELEMENTS, ACH)],
            a_buf.at[pl.ds(b * ACH, ACH)], sem))
        return cps

    def issue_in(i, b):
        @pl.when(valid(i))
        def _():
            for cp in in_copies(i, b):
                cp.start()

    def wait_in(i, b):
        @pl.when(valid(i))
        def _():
            for cp in in_copies(i, b):
                cp.wait()

    def out_copy(i, b):
        base = chunk_base(i)
        return pltpu.make_async_copy(
            out_buf.at[pl.ds(b * OCH, OCH)],
            out_hbm.at[pl.ds(base * OUT_DIM, OCH)], out_sems[b])

    def compute(b):
        xo = b * XB
        ao = b * ACH
        oo = b * OCH

        @plsc.parallel_loop(0, C // L, unroll=C // L)
        def grp_body(g):
            nv8 = ao + (g * L + iota) * NUM_ELEMENTS
            best = plsc.load_gather(a_buf, [nv8])
            ei = jnp.zeros((L,), jnp.int32)
            for e in range(1, NUM_ELEMENTS):
                ae = plsc.load_gather(a_buf, [nv8 + e])
                gt = ae > best
                best = jnp.where(gt, ae, best)
                ei = jnp.where(gt, jnp.full((L,), e, jnp.int32), ei)
            ei_buf[pl.ds(g * L, L)] = ei

        @plsc.parallel_loop(0, C, unroll=8)
        def node_body(n):
            ev = plsc.load_gather(ei_buf, [jnp.full((L,), n, jnp.int32)])
            ev512 = ev * (OUT_DIM * NUM_PATH)
            for k in range(OUT_DIM // L):
                acc = None
                for p in range(NUM_PATH):
                    xv = x_buf[pl.ds(xo + p * XCH + n * OUT_DIM + k * L, L)]
                    wv = plsc.load_gather(
                        w_buf, [ev512 + ((k * L + iota) * NUM_PATH + p)])
                    t = xv * wv
                    acc = t if acc is None else acc + t
                out_buf[pl.ds(oo + n * OUT_DIM + k * L, L)] = acc * ALPHA

    issue_in(0, 0)

    def outer_body(io, carry):
        for b in range(2):
            i = 2 * io + b
            wait_in(i, b)
            issue_in(i + 1, 1 - b)

            @pl.when((i >= 2) & valid(i - 2))
            def _():
                out_copy(i - 2, b).wait()

            @pl.when(valid(i))
            def _():
                compute(b)
                out_copy(i, b).start()
        return carry

    lax.fori_loop(0, OUTER, outer_body, 0)

    for last in (2 * OUTER - 2, 2 * OUTER - 1):
        @pl.when(valid(last))
        def _():
            out_copy(last, last % 2).wait()


def kernel(x, node_attrs, weights):
    mesh = plsc.VectorSubcoreMesh(core_axis_name="c", subcore_axis_name="s",
                                  num_cores=NC, num_subcores=NS)
    f = pl.kernel(
        _body,
        out_type=jax.ShapeDtypeStruct((N_NODES * OUT_DIM,), jnp.float32),
        mesh=mesh,
        compiler_params=pltpu.CompilerParams(needs_layout_passes=False),
        scratch_types=[
            pltpu.VMEM((2 * XB,), jnp.float32),
            pltpu.VMEM((2 * OCH,), jnp.float32),
            pltpu.VMEM((2 * ACH,), jnp.float32),
            pltpu.VMEM((WSZ,), jnp.float32),
            pltpu.VMEM((C,), jnp.int32),
            pltpu.SemaphoreType.DMA,
            pltpu.SemaphoreType.DMA,
            pltpu.SemaphoreType.DMA,
            pltpu.SemaphoreType.DMA,
        ],
    )
    out_flat = f(x.reshape(-1), node_attrs.reshape(-1), weights.reshape(-1))
    return out_flat.reshape(N_NODES, OUT_DIM)


# speedup vs baseline: 1.1210x; 1.1210x over previous
import jax
import jax.numpy as jnp
from jax import lax
from jax.experimental import pallas as pl
from jax.experimental.pallas import tpu as pltpu
from jax.experimental.pallas import tpu_sc as plsc
import numpy as np

NUM_PATH = 4
OUT_DIM = 128
NUM_ELEMENTS = 8
N_NODES = 100000
ALPHA = 1.0 / np.sqrt(float(NUM_PATH))

L = 16
NC, NS = 2, 16
NW = NC * NS
C = 64
NCHUNK = (N_NODES + C - 1) // C
ITERS = (NCHUNK + NW - 1) // NW
OUTER = (ITERS + 1) // 2
LAST_BASE = N_NODES - C
WSZ = NUM_ELEMENTS * OUT_DIM * NUM_PATH
XCH = C * OUT_DIM
XB = NUM_PATH * XCH
ACH = C * NUM_ELEMENTS
OCH = C * OUT_DIM


def _body(x_hbm, attrs_hbm, w_hbm, out_hbm,
          x_buf, out_buf, a_buf, w_buf, ei_buf,
          isem0, isem1, osem0, osem1):
    wid = lax.axis_index("c") * NS + lax.axis_index("s")
    pltpu.sync_copy(w_hbm, w_buf)
    iota = lax.iota(jnp.int32, L)
    in_sems = (isem0, isem1)
    out_sems = (osem0, osem1)

    def valid(i):
        return (i * NW + wid) < NCHUNK

    def chunk_base(i):
        return jnp.minimum((i * NW + wid) * C, LAST_BASE)

    def in_copies(i, b):
        base = chunk_base(i)
        sem = in_sems[b]
        cps = [pltpu.make_async_copy(
                   x_hbm.at[pl.ds(p * N_NODES * OUT_DIM + base * OUT_DIM, XCH)],
                   x_buf.at[pl.ds(b * XB + p * XCH, XCH)], sem)
               for p in range(NUM_PATH)]
        cps.append(pltpu.make_async_copy(
            attrs_hbm.at[pl.ds(base * NUM_ELEMENTS, ACH)],
            a_buf.at[pl.ds(b * ACH, ACH)], sem))
        return cps

    def issue_in(i, b):
        @pl.when(valid(i))
        def _():
            for cp in in_copies(i, b):
                cp.start()

    def wait_in(i, b):
        @pl.when(valid(i))
        def _():
            for cp in in_copies(i, b):
                cp.wait()

    def out_copy(i, b):
        base = chunk_base(i)
        return pltpu.make_async_copy(
            out_buf.at[pl.ds(b * OCH, OCH)],
            out_hbm.at[pl.ds(base * OUT_DIM, OCH)], out_sems[b])

    def compute(b):
        xo = b * XB
        ao = b * ACH
        oo = b * OCH

        @plsc.parallel_loop(0, C // L, unroll=C // L)
        def grp_body(g):
            nv8 = ao + (g * L + iota) * NUM_ELEMENTS
            best = plsc.load_gather(a_buf, [nv8])
            ei = jnp.zeros((L,), jnp.int32)
            for e in range(1, NUM_ELEMENTS):
                ae = plsc.load_gather(a_buf, [nv8 + e])
                gt = ae > best
                best = jnp.where(gt, ae, best)
                ei = jnp.where(gt, jnp.full((L,), e, jnp.int32), ei)
            ei_buf[pl.ds(g * L, L)] = ei

        @plsc.parallel_loop(0, C, unroll=4)
        def node_body(n):
            ev = plsc.load_gather(ei_buf, [jnp.full((L,), n, jnp.int32)])
            ev512 = ev * (OUT_DIM * NUM_PATH)
            for k in range(OUT_DIM // L):
                acc = None
                for p in range(NUM_PATH):
                    xv = x_buf[pl.ds(xo + p * XCH + n * OUT_DIM + k * L, L)]
                    wv = plsc.load_gather(
                        w_buf, [ev512 + (p * OUT_DIM + k * L) + iota])
                    t = xv * wv
                    acc = t if acc is None else acc + t
                out_buf[pl.ds(oo + n * OUT_DIM + k * L, L)] = acc * ALPHA

    issue_in(0, 0)

    def outer_body(io, carry):
        for b in range(2):
            i = 2 * io + b
            wait_in(i, b)
            issue_in(i + 1, 1 - b)

            @pl.when((i >= 2) & valid(i - 2))
            def _():
                out_copy(i - 2, b).wait()

            @pl.when(valid(i))
            def _():
                compute(b)
                out_copy(i, b).start()
        return carry

    lax.fori_loop(0, OUTER, outer_body, 0)

    for last in (2 * OUTER - 2, 2 * OUTER - 1):
        @pl.when(valid(last))
        def _():
            out_copy(last, last % 2).wait()


def kernel(x, node_attrs, weights):
    mesh = plsc.VectorSubcoreMesh(core_axis_name="c", subcore_axis_name="s",
                                  num_cores=NC, num_subcores=NS)
    f = pl.kernel(
        _body,
        out_type=jax.ShapeDtypeStruct((N_NODES * OUT_DIM,), jnp.float32),
        mesh=mesh,
        compiler_params=pltpu.CompilerParams(needs_layout_passes=False),
        scratch_types=[
            pltpu.VMEM((2 * XB,), jnp.float32),
            pltpu.VMEM((2 * OCH,), jnp.float32),
            pltpu.VMEM((2 * ACH,), jnp.float32),
            pltpu.VMEM((WSZ,), jnp.float32),
            pltpu.VMEM((C,), jnp.int32),
            pltpu.SemaphoreType.DMA,
            pltpu.SemaphoreType.DMA,
            pltpu.SemaphoreType.DMA,
            pltpu.SemaphoreType.DMA,
        ],
    )
    w_pm = jnp.transpose(weights, (0, 2, 1))
    out_flat = f(x.reshape(-1), node_attrs.reshape(-1), w_pm.reshape(-1))
    return out_flat.reshape(N_NODES, OUT_DIM)


# speedup vs baseline: 1.7944x; 1.6007x over previous
import jax
import jax.numpy as jnp
from jax import lax
from jax.experimental import pallas as pl
from jax.experimental.pallas import tpu as pltpu
from jax.experimental.pallas import tpu_sc as plsc
import numpy as np

NUM_PATH = 4
OUT_DIM = 128
NUM_ELEMENTS = 8
N_NODES = 100000
ALPHA = 1.0 / np.sqrt(float(NUM_PATH))

L = 16
NC, NS = 2, 16
NW = NC * NS
C = 64
NCHUNK = (N_NODES + C - 1) // C
ITERS = (NCHUNK + NW - 1) // NW
OUTER = (ITERS + 1) // 2
LAST_BASE = N_NODES - C
WSZ = NUM_ELEMENTS * OUT_DIM * NUM_PATH
XCH = C * OUT_DIM
XB = NUM_PATH * XCH
OCH = C * OUT_DIM


def _body(x_hbm, attrs_hbm, w_hbm, out_hbm,
          x_buf, out_buf, a_buf, w_buf, ei_vmem,
          isem0, isem1, osem0, osem1):
    wid = lax.axis_index("c") * NS + lax.axis_index("s")
    pltpu.sync_copy(w_hbm, w_buf)
    in_sems = (isem0, isem1)
    out_sems = (osem0, osem1)

    def valid(i):
        return (i * NW + wid) < NCHUNK

    def chunk_base(i):
        return jnp.minimum((i * NW + wid) * C, LAST_BASE)

    def in_copies(i, b):
        base = chunk_base(i)
        sem = in_sems[b]
        cps = [pltpu.make_async_copy(
                   x_hbm.at[pl.ds(p * N_NODES * OUT_DIM + base * OUT_DIM, XCH)],
                   x_buf.at[pl.ds(b * XB + p * XCH, XCH)], sem)
               for p in range(NUM_PATH)]
        cps.extend(pltpu.make_async_copy(
            attrs_hbm.at[pl.ds(e * N_NODES + base, C)], a_buf.at[b, e], sem)
            for e in range(NUM_ELEMENTS))
        return cps

    def issue_in(i, b):
        @pl.when(valid(i))
        def _():
            for cp in in_copies(i, b):
                cp.start()

    def wait_in(i, b):
        @pl.when(valid(i))
        def _():
            for cp in in_copies(i, b):
                cp.wait()

    def out_copy(i, b):
        base = chunk_base(i)
        return pltpu.make_async_copy(
            out_buf.at[pl.ds(b * OCH, OCH)],
            out_hbm.at[pl.ds(base * OUT_DIM, OCH)], out_sems[b])

    def compute(b):
        xo = b * XB
        oo = b * OCH

        @plsc.parallel_loop(0, C // L, unroll=C // L)
        def grp_body(g):
            sl = pl.ds(g * L, L)
            best = a_buf[b, 0, sl]
            ei = jnp.zeros((L,), jnp.int32)
            for e in range(1, NUM_ELEMENTS):
                ae = a_buf[b, e, sl]
                gt = ae > best
                best = jnp.where(gt, ae, best)
                ei = jnp.where(gt, jnp.full((L,), e, jnp.int32), ei)
            ei_vmem[sl] = ei

        @plsc.parallel_loop(0, C, unroll=4)
        def node_body(n):
            se = ei_vmem[pl.ds(n, L)][0]
            wb = se * (OUT_DIM * NUM_PATH)
            for k in range(OUT_DIM // L):
                acc = None
                for p in range(NUM_PATH):
                    xv = x_buf[pl.ds(xo + p * XCH + n * OUT_DIM + k * L, L)]
                    wv = w_buf[pl.ds(wb + p * OUT_DIM + k * L, L)]
                    t = xv * wv
                    acc = t if acc is None else acc + t
                out_buf[pl.ds(oo + n * OUT_DIM + k * L, L)] = acc * ALPHA

    issue_in(0, 0)

    def outer_body(io, carry):
        for b in range(2):
            i = 2 * io + b
            wait_in(i, b)
            issue_in(i + 1, 1 - b)

            @pl.when((i >= 2) & valid(i - 2))
            def _():
                out_copy(i - 2, b).wait()

            @pl.when(valid(i))
            def _():
                compute(b)
                out_copy(i, b).start()
        return carry

    lax.fori_loop(0, OUTER, outer_body, 0)

    for last in (2 * OUTER - 2, 2 * OUTER - 1):
        @pl.when(valid(last))
        def _():
            out_copy(last, last % 2).wait()


def kernel(x, node_attrs, weights):
    mesh = plsc.VectorSubcoreMesh(core_axis_name="c", subcore_axis_name="s",
                                  num_cores=NC, num_subcores=NS)
    f = pl.kernel(
        _body,
        out_type=jax.ShapeDtypeStruct((N_NODES * OUT_DIM,), jnp.float32),
        mesh=mesh,
        compiler_params=pltpu.CompilerParams(needs_layout_passes=False),
        scratch_types=[
            pltpu.VMEM((2 * XB,), jnp.float32),
            pltpu.VMEM((2 * OCH,), jnp.float32),
            pltpu.VMEM((2, NUM_ELEMENTS, C), jnp.float32),
            pltpu.VMEM((WSZ,), jnp.float32),
            pltpu.VMEM((C + L,), jnp.int32),
            pltpu.SemaphoreType.DMA,
            pltpu.SemaphoreType.DMA,
            pltpu.SemaphoreType.DMA,
            pltpu.SemaphoreType.DMA,
        ],
    )
    w_pm = jnp.transpose(weights, (0, 2, 1))
    out_flat = f(x.reshape(-1), jnp.transpose(node_attrs).reshape(-1),
                 w_pm.reshape(-1))
    return out_flat.reshape(N_NODES, OUT_DIM)
